# baseline (speedup 1.0000x reference)
import functools

import jax
import jax.numpy as jnp
from jax import lax
from jax.experimental import pallas as pl
from jax.experimental.pallas import tpu as pltpu
from jax.experimental.pallas import tpu_sc as plsc

N_NODES = 10000
N_EDGES = 320000
D = 128


_NW = 32
_PER_W = N_EDGES // _NW
_CH = 80
_N_CH = _PER_W // _CH

@functools.cache
def _make_sc_gather():
    mesh = plsc.VectorSubcoreMesh(core_axis_name="c", subcore_axis_name="s")
    n_pairs = (_N_CH - 1) // 2

    @functools.partial(
        pl.kernel,
        mesh=mesh,
        compiler_params=pltpu.CompilerParams(use_tc_tiling_on_sc=False),
        out_type=jax.ShapeDtypeStruct((N_EDGES, D // 2), jnp.int32),
        scratch_types=[
            pltpu.VMEM((_N_CH, _CH), jnp.int32),
            pltpu.VMEM((_CH, D // 2), jnp.int32),
            pltpu.VMEM((_CH, D // 2), jnp.int32),
            pltpu.SemaphoreType.DMA,
            pltpu.SemaphoreType.DMA,
            pltpu.SemaphoreType.DMA,
            pltpu.SemaphoreType.DMA,
        ],
    )
    def _sc_gather(table_hbm, idx_hbm, out_hbm, idx_v, rows0, rows1,
                   gs0, gs1, os0, os1):
        wid = lax.axis_index("s") * 2 + lax.axis_index("c")
        base = wid * _PER_W
        pltpu.sync_copy(idx_hbm.at[wid], idx_v)

        def g_start(i, buf, sem):
            pltpu.async_copy(table_hbm.at[idx_v.at[i]], buf, sem)

        def g_wait(buf, sem):
            pltpu.make_async_copy(table_hbm.at[idx_v.at[0]], buf, sem).wait()

        def o_start(i, buf, sem):
            pltpu.async_copy(buf, out_hbm.at[pl.ds(base + i * _CH, _CH)], sem)

        def o_wait(buf, sem):
            pltpu.make_async_copy(buf, out_hbm.at[pl.ds(base, _CH)], sem).wait()

        g_start(0, rows0, gs0)
        g_wait(rows0, gs0)
        o_start(0, rows0, os0)
        g_start(1, rows1, gs1)

        def body(g, carry):
            i1 = 2 * g + 1
            g_wait(rows1, gs1)
            o_start(i1, rows1, os1)
            o_wait(rows0, os0)
            g_start(i1 + 1, rows0, gs0)
            g_wait(rows0, gs0)
            o_start(i1 + 1, rows0, os0)
            o_wait(rows1, os1)
            @pl.when(g < n_pairs - 1)
            def _():
                g_start(i1 + 2, rows1, gs1)
            return carry

        lax.fori_loop(0, n_pairs, body, 0, unroll=False)
        o_wait(rows0, os0)

    return _sc_gather


_BLK = 6400


def _tc_body(e_ref, s_ref, m1_ref, c1l_ref, g1v_ref, sa_ref, sb_ref,
             ws_ref, j_ref, g2_ref, b2_ref, cst_ref, out_ref):
    f32 = jnp.float32
    e = e_ref[...]
    s = s_ref[...].astype(f32)
    m1 = jnp.dot(e, m1_ref[...], preferred_element_type=f32)
    yc = m1[:, :D] + c1l_ref[...]
    t = m1[:, D:D + 8]
    sq = yc * yc
    v1 = jnp.dot(sq, ws_ref[...], preferred_element_type=f32)
    ssc = jnp.dot(s, ws_ref[...], preferred_element_type=f32)

    T = jnp.concatenate([t, ssc, v1], axis=1).T
    c = cst_ref[...]
    mu1 = T[0:1] + c[0, 0]
    is1 = lax.rsqrt(T[19:20] + 1e-5)
    s0 = T[8:9] + c[0, 10]
    s1 = is1 * (T[1:2] + c[0, 1] - mu1 * c[0, 2]) + c[0, 3]
    hu = is1 * (T[2:3] + c[0, 4] - mu1 * c[0, 5]) + c[0, 6]
    mhe = is1 * (T[3:4] + c[0, 7] - mu1 * c[0, 8]) + c[0, 9]
    l0 = jnp.where(s0 >= 0, s0, 0.01 * s0)
    l1 = jnp.where(s1 >= 0, s1, 0.01 * s1)
    a1 = 1.0 / (1.0 + jnp.exp(l0 - l1))
    a0 = 1.0 - a1
    bl = a0 * T[9:10] + a1 * hu + T[10:11]
    beta = 1.0 / (1.0 + jnp.exp(-bl))
    c2 = (1.0 - beta) * a1
    c1 = 1.0 - c2
    d2 = c2 * is1
    mu2 = c1 * T[11:12] + c2 * mhe
    coef = jnp.concatenate([c1, d2, c2, mu2, mu2, mu2, mu2, mu2], axis=0).T

    ca = jnp.dot(coef, sa_ref[...], preferred_element_type=f32)
    cb = jnp.dot(coef, sb_ref[...], preferred_element_type=f32)
    z = yc * g1v_ref[...]
    hc = ca[:, :D] * s + ca[:, D:] * z + cb
    sq2 = hc * hc
    var2 = jnp.dot(sq2, j_ref[...], preferred_element_type=f32)
    o = hc * (lax.rsqrt(var2 + 1e-5) * g2_ref[...]) + b2_ref[...]
    out_ref[...] = jnp.maximum(o, 0.0)


def _tc_dense(edge_embedding, src_h, m1, c1l, g1v, sa, sb, ws, j, g2, b2,
              cst):
    n_blk = N_EDGES // _BLK
    full = pl.BlockSpec((1, D), lambda i: (0, 0))
    return pl.pallas_call(
        _tc_body,
        grid=(n_blk,),
        in_specs=[
            pl.BlockSpec((_BLK, 16), lambda i: (i, 0)),
            pl.BlockSpec((_BLK, D), lambda i: (i, 0)),
            pl.BlockSpec((16, D + 8), lambda i: (0, 0)),
            full, full,
            pl.BlockSpec((8, 2 * D), lambda i: (0, 0)),
            pl.BlockSpec((8, D), lambda i: (0, 0)),
            pl.BlockSpec((D, 8), lambda i: (0, 0)),
            pl.BlockSpec((D, D), lambda i: (0, 0)),
            full, full,
            pl.BlockSpec((1, 16), lambda i: (0, 0)),
        ],
        out_specs=pl.BlockSpec((_BLK, D), lambda i: (i, 0)),
        out_shape=jax.ShapeDtypeStruct((N_EDGES, D), jnp.float32),
    )(edge_embedding, src_h, m1, c1l, g1v, sa, sb, ws, j, g2, b2, cst)


def _prep_params(lin1_W, lin1_b, ln1_g, ln1_b, attn_W, attn_b, beta_W,
                 lne_g, lne_b):
    f32 = jnp.float32
    w = attn_W[:, 0].astype(f32)
    ab = attn_b[0].astype(f32)
    bw = beta_W[:, 0].astype(f32)
    u = bw[:D] + bw[2 * D:]
    v = bw[D:2 * D] - bw[2 * D:]
    W1, b1, g1, bb1 = lin1_W, lin1_b, ln1_g, ln1_b

    w1_rowmean = jnp.mean(W1, axis=1, keepdims=True)
    mb1 = jnp.mean(b1)
    g1w = g1 * w
    g1u = g1 * u
    g1m = g1 / float(D)
    extra = jnp.stack(
        [w1_rowmean[:, 0], W1 @ g1w, W1 @ g1u, W1 @ g1m], axis=1)
    m1 = jnp.concatenate(
        [W1 - w1_rowmean, extra, jnp.zeros((16, 4), f32)], axis=1)
    c1l = (b1 - mb1).reshape(1, D)

    ones_d = jnp.full((D,), 1.0 / D, f32)
    ws = jnp.stack([w, u, v, ones_d], axis=1)
    ws = jnp.concatenate([ws, jnp.zeros((D, 4), f32)], axis=1)
    j = jnp.full((D, D), 1.0 / D, f32)

    one_row = jnp.ones((1, D), f32)
    zero_row = jnp.zeros((1, D), f32)
    sa = jnp.concatenate([
        jnp.concatenate([one_row, zero_row], axis=1),
        jnp.concatenate([zero_row, one_row], axis=1),
        jnp.zeros((6, 2 * D), f32),
    ], axis=0)
    sb = jnp.concatenate([
        zero_row, zero_row,
        bb1.reshape(1, D),
        -one_row,
        jnp.zeros((4, D), f32),
    ], axis=0)

    cst = jnp.stack([
        mb1,
        jnp.dot(b1, g1w),
        jnp.sum(g1w),
        jnp.dot(bb1, w) + ab,
        jnp.dot(b1, g1u),
        jnp.sum(g1u),
        jnp.dot(bb1, u),
        jnp.dot(b1, g1m),
        jnp.mean(g1),
        jnp.mean(bb1),
        ab,
        0.0, 0.0, 0.0, 0.0, 0.0,
    ]).reshape(1, 16)

    r1 = lambda x: x.reshape(1, D)
    return (m1, c1l, r1(g1), sa, sb, ws, j, r1(lne_g), r1(lne_b), cst)


def kernel(atom_embedding, edge_embedding, edge_index,
           lin1_W, lin1_b, ln1_g, ln1_b,
           attn_W, attn_b, beta_W, lne_g, lne_b):
    src = edge_index[0].astype(jnp.int32).reshape(_NW, _N_CH, _CH)
    table_i = lax.bitcast_convert_type(
        atom_embedding.astype(jnp.bfloat16).reshape(N_NODES, D // 2, 2),
        jnp.int32)
    src_g = _make_sc_gather()(table_i, src)
    src_h = lax.bitcast_convert_type(src_g, jnp.bfloat16).reshape(N_EDGES, D)
    prep = _prep_params(lin1_W, lin1_b, ln1_g, ln1_b, attn_W, attn_b,
                        beta_W, lne_g, lne_b)
    return _tc_dense(edge_embedding, src_h, *prep)


# speedup vs baseline: 2.6606x; 2.6606x over previous
import functools

import jax
import jax.numpy as jnp
from jax import lax
from jax.experimental import pallas as pl
from jax.experimental.pallas import tpu as pltpu
from jax.experimental.pallas import tpu_sc as plsc

N_NODES = 10000
N_EDGES = 320000
D = 128


_NW = 32


@functools.cache
def _make_sc_gather(n_edges, ch):
    per_w = n_edges // _NW
    n_ch = per_w // ch
    mesh = plsc.VectorSubcoreMesh(core_axis_name="c", subcore_axis_name="s")
    n_pairs = (n_ch - 1) // 2

    @functools.partial(
        pl.kernel,
        mesh=mesh,
        out_type=jax.ShapeDtypeStruct((n_edges, D), jnp.float32),
        scratch_types=[
            pltpu.VMEM((n_ch, ch), jnp.int32),
            pltpu.VMEM((ch, D), jnp.float32),
            pltpu.VMEM((ch, D), jnp.float32),
            pltpu.SemaphoreType.DMA,
            pltpu.SemaphoreType.DMA,
            pltpu.SemaphoreType.DMA,
            pltpu.SemaphoreType.DMA,
        ],
    )
    def _sc_gather(table_hbm, idx_hbm, out_hbm, idx_v, rows0, rows1,
                   gs0, gs1, os0, os1):
        _CH = ch
        wid = lax.axis_index("s") * 2 + lax.axis_index("c")
        base = wid * per_w
        pltpu.sync_copy(idx_hbm.at[wid], idx_v)

        def g_start(i, buf, sem):
            pltpu.async_copy(table_hbm.at[idx_v.at[i]], buf, sem)

        def g_wait(buf, sem):
            pltpu.make_async_copy(table_hbm.at[idx_v.at[0]], buf, sem).wait()

        def o_start(i, buf, sem):
            pltpu.async_copy(buf, out_hbm.at[pl.ds(base + i * _CH, _CH)], sem)

        def o_wait(buf, sem):
            pltpu.make_async_copy(buf, out_hbm.at[pl.ds(base, _CH)], sem).wait()

        g_start(0, rows0, gs0)
        g_wait(rows0, gs0)
        o_start(0, rows0, os0)
        g_start(1, rows1, gs1)

        def body(g, carry):
            i1 = 2 * g + 1
            g_wait(rows1, gs1)
            o_start(i1, rows1, os1)
            o_wait(rows0, os0)
            g_start(i1 + 1, rows0, gs0)
            g_wait(rows0, gs0)
            o_start(i1 + 1, rows0, os0)
            o_wait(rows1, os1)
            @pl.when(g < n_pairs - 1)
            def _():
                g_start(i1 + 2, rows1, gs1)
            return carry

        lax.fori_loop(0, n_pairs, body, 0, unroll=False)
        o_wait(rows0, os0)

    return _sc_gather


_BLK = 6400


def _tc_body(e_ref, s_ref, m1_ref, c1l_ref, g1v_ref, sa_ref, sb_ref,
             ws_ref, j_ref, g2_ref, b2_ref, cst_ref, *rest_refs):
    out_ref = rest_refs[-1]
    f32 = jnp.float32
    e = e_ref[...]
    s = s_ref[...]
    m1 = jnp.dot(e, m1_ref[...], preferred_element_type=f32)
    yc = m1[:, :D] + c1l_ref[...]
    t = m1[:, D:D + 8]
    sq = yc * yc
    v1 = jnp.dot(sq, ws_ref[...], preferred_element_type=f32)
    ssc = jnp.dot(s, ws_ref[...], preferred_element_type=f32)

    T = jnp.concatenate([t, ssc, v1], axis=1).T
    c = cst_ref[...]
    mu1 = T[0:1] + c[0, 0]
    is1 = lax.rsqrt(T[19:20] + 1e-5)
    s0 = T[8:9] + c[0, 10]
    s1 = is1 * (T[1:2] + c[0, 1] - mu1 * c[0, 2]) + c[0, 3]
    hu = is1 * (T[2:3] + c[0, 4] - mu1 * c[0, 5]) + c[0, 6]
    mhe = is1 * (T[3:4] + c[0, 7] - mu1 * c[0, 8]) + c[0, 9]
    l0 = jnp.where(s0 >= 0, s0, 0.01 * s0)
    l1 = jnp.where(s1 >= 0, s1, 0.01 * s1)
    a1 = 1.0 / (1.0 + jnp.exp(l0 - l1))
    a0 = 1.0 - a1
    bl = a0 * T[9:10] + a1 * hu + T[10:11]
    beta = 1.0 / (1.0 + jnp.exp(-bl))
    c2 = (1.0 - beta) * a1
    c1 = 1.0 - c2
    d2 = c2 * is1
    mu2 = c1 * T[11:12] + c2 * mhe
    coef = jnp.concatenate([c1, d2, c2, mu2, mu2, mu2, mu2, mu2], axis=0).T

    ca = jnp.dot(coef, sa_ref[...], preferred_element_type=f32)
    cb = jnp.dot(coef, sb_ref[...], preferred_element_type=f32)
    z = yc * g1v_ref[...]
    hc = ca[:, :D] * s + ca[:, D:] * z + cb
    sq2 = hc * hc
    var2 = jnp.dot(sq2, j_ref[...], preferred_element_type=f32)
    o = hc * (lax.rsqrt(var2 + 1e-5) * g2_ref[...]) + b2_ref[...]
    out_ref[...] = jnp.maximum(o, 0.0)


def _tc_dense_sliced(edge_embedding, srch_halves, prep):
    n_half = N_EDGES // 2 // _BLK
    full = pl.BlockSpec((1, D), lambda i: (0, 0))

    def specs(blk_off):
        return [
            pl.BlockSpec((_BLK, 16), lambda i: (i + blk_off, 0)),
            pl.BlockSpec((_BLK, D), lambda i: (i, 0)),
            pl.BlockSpec((16, D + 8), lambda i: (0, 0)),
            full, full,
            pl.BlockSpec((8, 2 * D), lambda i: (0, 0)),
            pl.BlockSpec((8, D), lambda i: (0, 0)),
            pl.BlockSpec((D, 8), lambda i: (0, 0)),
            pl.BlockSpec((D, D), lambda i: (0, 0)),
            full, full,
            pl.BlockSpec((1, 16), lambda i: (0, 0)),
        ]

    out_shape = jax.ShapeDtypeStruct((N_EDGES, D), jnp.float32)
    out1 = pl.pallas_call(
        _tc_body,
        grid=(n_half,),
        in_specs=specs(0),
        out_specs=pl.BlockSpec((_BLK, D), lambda i: (i, 0)),
        out_shape=out_shape,
    )(edge_embedding, srch_halves[0], *prep)
    return pl.pallas_call(
        _tc_body,
        grid=(n_half,),
        in_specs=specs(n_half) + [pl.BlockSpec(memory_space=pl.ANY)],
        out_specs=pl.BlockSpec((_BLK, D), lambda i: (i + n_half, 0)),
        out_shape=out_shape,
        input_output_aliases={12: 0},
    )(edge_embedding, srch_halves[1], *prep, out1)


def _prep_params(lin1_W, lin1_b, ln1_g, ln1_b, attn_W, attn_b, beta_W,
                 lne_g, lne_b):
    f32 = jnp.float32
    w = attn_W[:, 0].astype(f32)
    ab = attn_b[0].astype(f32)
    bw = beta_W[:, 0].astype(f32)
    u = bw[:D] + bw[2 * D:]
    v = bw[D:2 * D] - bw[2 * D:]
    W1, b1, g1, bb1 = lin1_W, lin1_b, ln1_g, ln1_b

    w1_rowmean = jnp.mean(W1, axis=1, keepdims=True)
    mb1 = jnp.mean(b1)
    g1w = g1 * w
    g1u = g1 * u
    g1m = g1 / float(D)
    extra = jnp.stack(
        [w1_rowmean[:, 0], W1 @ g1w, W1 @ g1u, W1 @ g1m], axis=1)
    m1 = jnp.concatenate(
        [W1 - w1_rowmean, extra, jnp.zeros((16, 4), f32)], axis=1)
    c1l = (b1 - mb1).reshape(1, D)

    ones_d = jnp.full((D,), 1.0 / D, f32)
    ws = jnp.stack([w, u, v, ones_d], axis=1)
    ws = jnp.concatenate([ws, jnp.zeros((D, 4), f32)], axis=1)
    j = jnp.full((D, D), 1.0 / D, f32)

    one_row = jnp.ones((1, D), f32)
    zero_row = jnp.zeros((1, D), f32)
    sa = jnp.concatenate([
        jnp.concatenate([one_row, zero_row], axis=1),
        jnp.concatenate([zero_row, one_row], axis=1),
        jnp.zeros((6, 2 * D), f32),
    ], axis=0)
    sb = jnp.concatenate([
        zero_row, zero_row,
        bb1.reshape(1, D),
        -one_row,
        jnp.zeros((4, D), f32),
    ], axis=0)

    cst = jnp.stack([
        mb1,
        jnp.dot(b1, g1w),
        jnp.sum(g1w),
        jnp.dot(bb1, w) + ab,
        jnp.dot(b1, g1u),
        jnp.sum(g1u),
        jnp.dot(bb1, u),
        jnp.dot(b1, g1m),
        jnp.mean(g1),
        jnp.mean(bb1),
        ab,
        0.0, 0.0, 0.0, 0.0, 0.0,
    ]).reshape(1, 16)

    r1 = lambda x: x.reshape(1, D)
    return (m1, c1l, r1(g1), sa, sb, ws, j, r1(lne_g), r1(lne_b), cst)


def kernel(atom_embedding, edge_embedding, edge_index,
           lin1_W, lin1_b, ln1_g, ln1_b,
           attn_W, attn_b, beta_W, lne_g, lne_b):
    half = N_EDGES // 2
    ch = 40
    n_ch = half // _NW // ch
    src = edge_index[0].astype(jnp.int32)
    gather = _make_sc_gather(half, ch)
    srch = [gather(atom_embedding, src[k * half:(k + 1) * half]
                   .reshape(_NW, n_ch, ch)) for k in (0, 1)]
    prep = _prep_params(lin1_W, lin1_b, ln1_g, ln1_b, attn_W, attn_b,
                        beta_W, lne_g, lne_b)
    return _tc_dense_sliced(edge_embedding, srch, prep)
